# baseline (speedup 1.0000x reference)
import numpy as np
import jax
import jax.numpy as jnp
from jax.experimental import pallas as pl
from jax.experimental.pallas import tpu as pltpu

_FEAT_STRIDE = 16
_PRE_NMS_TOP_N = 6000
_POST_NMS_TOP_N = 300
_NMS_THRESH = 0.7
_B = 4
_BB = 4
_G = _B // _BB
_FH = 64
_FW = 64
_A = 9
_N = _FH * _FW * _A
_LANES = 128
_ROWS = _N // _LANES


def _np_anchors():
    def whctrs(anchor):
        w = anchor[2] - anchor[0] + 1
        h = anchor[3] - anchor[1] + 1
        return w, h, anchor[0] + 0.5 * (w - 1), anchor[1] + 0.5 * (h - 1)

    def mkanchors(ws, hs, x_ctr, y_ctr):
        ws = ws[:, None]
        hs = hs[:, None]
        return np.hstack((x_ctr - 0.5 * (ws - 1), y_ctr - 0.5 * (hs - 1),
                          x_ctr + 0.5 * (ws - 1), y_ctr + 0.5 * (hs - 1)))

    ratios = np.array([0.5, 1.0, 2.0])
    scales = np.array([8, 16, 32])
    base = np.array([1, 1, 16, 16], dtype=np.float64) - 1
    w, h, xc, yc = whctrs(base)
    size_ratios = (w * h) / ratios
    ws = np.round(np.sqrt(size_ratios))
    hs = np.round(ws * ratios)
    ranchors = mkanchors(ws, hs, xc, yc)
    outs = []
    for i in range(ranchors.shape[0]):
        w, h, xc, yc = whctrs(ranchors[i])
        outs.append(mkanchors(w * scales, h * scales, xc, yc))
    return np.vstack(outs).astype(np.float32)


def _np_shifted_anchors():
    a9 = _np_anchors()
    sx = (np.arange(_FW, dtype=np.float32) * _FEAT_STRIDE)
    sy = (np.arange(_FH, dtype=np.float32) * _FEAT_STRIDE)
    syg, sxg = np.meshgrid(sy, sx, indexing="ij")
    shifts = np.stack([sxg.ravel(), syg.ravel(), sxg.ravel(), syg.ravel()],
                      axis=1).astype(np.float32)
    allanch = (a9[None, :, :] + shifts[:, None, :]).reshape(_N, 4)
    return [allanch[:, c].reshape(_ROWS, _LANES).copy() for c in range(4)]


_ANCH_PLANES = _np_shifted_anchors()


def _proposal_kernel(sc_ref, dx_ref, dy_ref, dw_ref, dh_ref,
                     ax1_ref, ay1_ref, ax2_ref, ay2_ref, im_ref,
                     out_ref,
                     x1s, y1s, x2s, y2s, ars, msr, outs):
    f32 = jnp.float32
    i32 = jnp.int32

    sc = sc_ref[...]
    ax1 = ax1_ref[...][None, :, :]
    ay1 = ay1_ref[...][None, :, :]
    ax2 = ax2_ref[...][None, :, :]
    ay2 = ay2_ref[...][None, :, :]

    widths = ax2 - ax1 + 1.0
    heights = ay2 - ay1 + 1.0
    ctr_x = ax1 + 0.5 * widths
    ctr_y = ay1 + 0.5 * heights
    pred_cx = dx_ref[...] * widths + ctr_x
    pred_cy = dy_ref[...] * heights + ctr_y
    pred_w = jnp.exp(dw_ref[...]) * widths
    pred_h = jnp.exp(dh_ref[...]) * heights
    x1 = pred_cx - 0.5 * pred_w
    y1 = pred_cy - 0.5 * pred_h
    x2 = pred_cx + 0.5 * pred_w
    y2 = pred_cy + 0.5 * pred_h

    imh = im_ref[:, 0:1, 0:1]
    imw = im_ref[:, 1:2, 0:1]
    x1 = jnp.minimum(jnp.maximum(x1, 0.0), imw - 1.0)
    y1 = jnp.minimum(jnp.maximum(y1, 0.0), imh - 1.0)
    x2 = jnp.minimum(jnp.maximum(x2, 0.0), imw - 1.0)
    y2 = jnp.minimum(jnp.maximum(y2, 0.0), imh - 1.0)
    area = (x2 - x1) * (y2 - y1)

    x1s[...] = x1
    y1s[...] = y1
    x2s[...] = x2
    y2s[...] = y2
    ars[...] = area

    def red_min(v):
        return jnp.min(jnp.min(v, axis=2, keepdims=True), axis=1, keepdims=True)

    def red_max(v):
        return jnp.max(jnp.max(v, axis=2, keepdims=True), axis=1, keepdims=True)

    def red_sum(v):
        return jnp.sum(jnp.sum(v, axis=2, keepdims=True), axis=1, keepdims=True)

    flat = (jax.lax.broadcasted_iota(i32, (_BB, _ROWS, _LANES), 1) * _LANES
            + jax.lax.broadcasted_iota(i32, (_BB, _ROWS, _LANES), 2))

    bits = jax.lax.bitcast_convert_type(sc, i32)

    def bs_body(_, st):
        lo, hi = st
        mid = (lo + hi) >> 1
        cnt = red_sum((bits >= mid).astype(i32))
        ge = cnt >= _PRE_NMS_TOP_N
        return jnp.where(ge, mid, lo), jnp.where(ge, hi, mid)

    lo0 = jnp.zeros((_BB, 1, 1), i32)
    hi0 = jnp.full((_BB, 1, 1), 0x3F800000, i32)
    tau, _ = jax.lax.fori_loop(0, 30, bs_body, (lo0, hi0))

    eq = bits == tau
    cnt_gt = red_sum((bits > tau).astype(i32))
    k_rem = _PRE_NMS_TOP_N - cnt_gt

    def bs2_body(_, st):
        lo2, hi2 = st
        mid = (lo2 + hi2) >> 1
        cnt = red_sum((eq & (flat <= mid)).astype(i32))
        geq = cnt >= k_rem
        return jnp.where(geq, lo2, mid), jnp.where(geq, mid, hi2)

    lo20 = jnp.full((_BB, 1, 1), -1, i32)
    hi20 = jnp.full((_BB, 1, 1), _N - 1, i32)
    _, mstar = jax.lax.fori_loop(0, 17, bs2_body, (lo20, hi20))

    valid0 = (bits > tau) | (eq & (flat <= mstar))
    ms0 = jnp.where(valid0, sc, -1.0)
    msr[...] = ms0

    lane_iota = jax.lax.broadcasted_iota(i32, (1, _LANES), 1)
    row_iota = jax.lax.broadcasted_iota(i32, (_BB, _ROWS, 1), 1)
    zero8 = [jnp.float32(0.0)] * 4

    def nms_body(j, ms):
        rm = jnp.max(ms, axis=2, keepdims=True)
        mx = jnp.max(rm, axis=1, keepdims=True)
        has = mx > -0.5
        ridx = jnp.min(jnp.where(rm == mx, row_iota, _ROWS),
                       axis=1, keepdims=True)[:, 0, 0]

        picked = []
        lanes = []
        for b in range(_BB):
            r = ridx[b]
            msrow = msr[b, pl.ds(r, 1), :]
            lane = jnp.min(jnp.where(msrow == mx[b, 0, 0], lane_iota, _LANES))
            lanes.append(r * _LANES + lane)
            lm = lane_iota == lane
            vals = []
            for ref in (x1s, y1s, x2s, y2s, ars):
                rowv = ref[b, pl.ds(r, 1), :]
                vals.append(jnp.sum(jnp.where(lm, rowv, 0.0)))
            picked.append(vals)

        idx = jnp.where(has[:, 0, 0], jnp.stack(lanes), _N)
        thr = jnp.where(has, _NMS_THRESH, 2.0)

        def bb(c):
            return jnp.stack([picked[b][c] for b in range(_BB)]).reshape(_BB, 1, 1)

        px1, py1, px2, py2, par = bb(0), bb(1), bb(2), bb(3), bb(4)

        xx1 = jnp.maximum(px1, x1)
        yy1 = jnp.maximum(py1, y1)
        xx2 = jnp.minimum(px2, x2)
        yy2 = jnp.minimum(py2, y2)
        inter = jnp.maximum(xx2 - xx1, 0.0) * jnp.maximum(yy2 - yy1, 0.0)
        iou = inter / (par + area - inter + 1e-9)
        kill = (iou > thr) | (flat == idx.reshape(_BB, 1, 1))
        ms = jnp.where(kill, -1.0, ms)
        msr[...] = ms

        hasf = has.astype(f32)[:, :, 0]
        rows = jnp.stack([
            jnp.stack([picked[b][0], picked[b][1],
                       picked[b][2], picked[b][3]] + zero8)
            for b in range(_BB)
        ])
        vals8 = rows[:, None, :] * hasf[:, :, None]
        outs[:, pl.ds(j, 1), :] = vals8
        return ms

    jax.lax.fori_loop(0, _POST_NMS_TOP_N, nms_body, ms0)

    col0 = (jax.lax.broadcasted_iota(i32, (_BB, _POST_NMS_TOP_N, 1), 0)
            + pl.program_id(0) * _BB).astype(f32)
    out_ref[...] = jnp.concatenate([col0, outs[:, 0:_POST_NMS_TOP_N, 0:4]],
                                   axis=2)


@jax.jit
def _impl(scores, bbox_deltas, im_info):
    sc = scores.transpose(0, 2, 3, 1).reshape(_B, _ROWS, _LANES)
    d = bbox_deltas.transpose(0, 2, 3, 1).reshape(_B, _N, 4)
    dx = d[..., 0].reshape(_B, _ROWS, _LANES)
    dy = d[..., 1].reshape(_B, _ROWS, _LANES)
    dw = d[..., 2].reshape(_B, _ROWS, _LANES)
    dh = d[..., 3].reshape(_B, _ROWS, _LANES)
    imf = jnp.broadcast_to(im_info[:, :2, None], (_B, 2, _LANES))
    imf = jnp.concatenate([imf, jnp.zeros((_B, 6, _LANES), jnp.float32)], axis=1)

    a1, a2, a3, a4 = (jnp.asarray(p) for p in _ANCH_PLANES)

    bspec = pl.BlockSpec((_BB, _ROWS, _LANES), lambda i: (i, 0, 0))
    aspec = pl.BlockSpec((_ROWS, _LANES), lambda i: (0, 0))
    return pl.pallas_call(
        _proposal_kernel,
        grid=(_G,),
        in_specs=[bspec] * 5 + [aspec] * 4
                 + [pl.BlockSpec((_BB, 8, _LANES), lambda i: (i, 0, 0))],
        out_specs=pl.BlockSpec((_BB, _POST_NMS_TOP_N, 5), lambda i: (i, 0, 0)),
        out_shape=jax.ShapeDtypeStruct((_B, _POST_NMS_TOP_N, 5), jnp.float32),
        scratch_shapes=[pltpu.VMEM((_BB, _ROWS, _LANES), jnp.float32)
                        for _ in range(6)]
                       + [pltpu.VMEM((_BB, 304, 8), jnp.float32)],
        compiler_params=pltpu.CompilerParams(
            dimension_semantics=("parallel",)),
    )(sc, dx, dy, dw, dh, a1, a2, a3, a4, imf)


def kernel(scores, bbox_deltas, im_info):
    return _impl(scores, bbox_deltas, im_info)


# speedup vs baseline: 1.2268x; 1.2268x over previous
import numpy as np
import jax
import jax.numpy as jnp
from jax.experimental import pallas as pl
from jax.experimental.pallas import tpu as pltpu

_FEAT_STRIDE = 16
_PRE_NMS_TOP_N = 6000
_POST_NMS_TOP_N = 300
_NMS_THRESH = 0.7
_B = 4
_BB = 4
_G = _B // _BB
_FH = 64
_FW = 64
_A = 9
_N = _FH * _FW * _A
_LANES = 128
_ROWS = _N // _LANES


def _np_anchors():
    def whctrs(anchor):
        w = anchor[2] - anchor[0] + 1
        h = anchor[3] - anchor[1] + 1
        return w, h, anchor[0] + 0.5 * (w - 1), anchor[1] + 0.5 * (h - 1)

    def mkanchors(ws, hs, x_ctr, y_ctr):
        ws = ws[:, None]
        hs = hs[:, None]
        return np.hstack((x_ctr - 0.5 * (ws - 1), y_ctr - 0.5 * (hs - 1),
                          x_ctr + 0.5 * (ws - 1), y_ctr + 0.5 * (hs - 1)))

    ratios = np.array([0.5, 1.0, 2.0])
    scales = np.array([8, 16, 32])
    base = np.array([1, 1, 16, 16], dtype=np.float64) - 1
    w, h, xc, yc = whctrs(base)
    size_ratios = (w * h) / ratios
    ws = np.round(np.sqrt(size_ratios))
    hs = np.round(ws * ratios)
    ranchors = mkanchors(ws, hs, xc, yc)
    outs = []
    for i in range(ranchors.shape[0]):
        w, h, xc, yc = whctrs(ranchors[i])
        outs.append(mkanchors(w * scales, h * scales, xc, yc))
    return np.vstack(outs).astype(np.float32)


def _np_shifted_anchors():
    a9 = _np_anchors()
    sx = (np.arange(_FW, dtype=np.float32) * _FEAT_STRIDE)
    sy = (np.arange(_FH, dtype=np.float32) * _FEAT_STRIDE)
    syg, sxg = np.meshgrid(sy, sx, indexing="ij")
    shifts = np.stack([sxg.ravel(), syg.ravel(), sxg.ravel(), syg.ravel()],
                      axis=1).astype(np.float32)
    allanch = (a9[None, :, :] + shifts[:, None, :]).reshape(_N, 4)
    return [allanch[:, c].reshape(_ROWS, _LANES).copy() for c in range(4)]


_ANCH_PLANES = _np_shifted_anchors()


def _proposal_kernel(sc_ref, dx_ref, dy_ref, dw_ref, dh_ref,
                     ax1_ref, ay1_ref, ax2_ref, ay2_ref, im_ref,
                     out_ref,
                     x1s, y1s, x2s, y2s, ars, outs):
    f32 = jnp.float32
    i32 = jnp.int32

    sc = sc_ref[...]
    ax1 = ax1_ref[...][None, :, :]
    ay1 = ay1_ref[...][None, :, :]
    ax2 = ax2_ref[...][None, :, :]
    ay2 = ay2_ref[...][None, :, :]

    widths = ax2 - ax1 + 1.0
    heights = ay2 - ay1 + 1.0
    ctr_x = ax1 + 0.5 * widths
    ctr_y = ay1 + 0.5 * heights
    pred_cx = dx_ref[...] * widths + ctr_x
    pred_cy = dy_ref[...] * heights + ctr_y
    pred_w = jnp.exp(dw_ref[...]) * widths
    pred_h = jnp.exp(dh_ref[...]) * heights
    x1 = pred_cx - 0.5 * pred_w
    y1 = pred_cy - 0.5 * pred_h
    x2 = pred_cx + 0.5 * pred_w
    y2 = pred_cy + 0.5 * pred_h

    imh = im_ref[:, 0:1, 0:1]
    imw = im_ref[:, 1:2, 0:1]
    x1 = jnp.minimum(jnp.maximum(x1, 0.0), imw - 1.0)
    y1 = jnp.minimum(jnp.maximum(y1, 0.0), imh - 1.0)
    x2 = jnp.minimum(jnp.maximum(x2, 0.0), imw - 1.0)
    y2 = jnp.minimum(jnp.maximum(y2, 0.0), imh - 1.0)
    area = (x2 - x1) * (y2 - y1)

    x1s[...] = x1
    y1s[...] = y1
    x2s[...] = x2
    y2s[...] = y2
    ars[...] = area

    def red_min(v):
        return jnp.min(jnp.min(v, axis=2, keepdims=True), axis=1, keepdims=True)

    def red_max(v):
        return jnp.max(jnp.max(v, axis=2, keepdims=True), axis=1, keepdims=True)

    def red_sum(v):
        return jnp.sum(jnp.sum(v, axis=2, keepdims=True), axis=1, keepdims=True)

    flat = (jax.lax.broadcasted_iota(i32, (_BB, _ROWS, _LANES), 1) * _LANES
            + jax.lax.broadcasted_iota(i32, (_BB, _ROWS, _LANES), 2))

    bits = jax.lax.bitcast_convert_type(sc, i32)

    def bs_body(_, st):
        lo, hi = st
        mid = (lo + hi) >> 1
        cnt = red_sum((bits >= mid).astype(i32))
        ge = cnt >= _PRE_NMS_TOP_N
        return jnp.where(ge, mid, lo), jnp.where(ge, hi, mid)

    lo0 = jnp.zeros((_BB, 1, 1), i32)
    hi0 = jnp.full((_BB, 1, 1), 0x3F800000, i32)
    tau, _ = jax.lax.fori_loop(0, 30, bs_body, (lo0, hi0))

    eq = bits == tau
    cnt_gt = red_sum((bits > tau).astype(i32))
    k_rem = _PRE_NMS_TOP_N - cnt_gt

    def bs2_body(_, st):
        lo2, hi2 = st
        mid = (lo2 + hi2) >> 1
        cnt = red_sum((eq & (flat <= mid)).astype(i32))
        geq = cnt >= k_rem
        return jnp.where(geq, lo2, mid), jnp.where(geq, mid, hi2)

    lo20 = jnp.full((_BB, 1, 1), -1, i32)
    hi20 = jnp.full((_BB, 1, 1), _N - 1, i32)
    _, mstar = jax.lax.fori_loop(0, 17, bs2_body, (lo20, hi20))

    valid0 = (bits > tau) | (eq & (flat <= mstar))
    ms0 = jnp.where(valid0, sc, -1.0)

    lane_iota = jax.lax.broadcasted_iota(i32, (1, _LANES), 1)
    zero8 = [jnp.float32(0.0)] * 4

    def nms_body(j, ms):
        mx = red_max(ms)
        has = mx > -0.5
        idx = red_min(jnp.where(ms == mx, flat, _N))[:, 0, 0]
        idx = jnp.where(has[:, 0, 0], idx, _N)
        thr = jnp.where(has, _NMS_THRESH, 2.0)

        picked = []
        for b in range(_BB):
            r = jnp.minimum(idx[b] >> 7, _ROWS - 1)
            l = idx[b] & 127
            lm = lane_iota == l
            vals = []
            for ref in (x1s, y1s, x2s, y2s, ars):
                rowv = ref[b, pl.ds(r, 1), :]
                vals.append(jnp.sum(jnp.where(lm, rowv, 0.0)))
            picked.append(vals)

        def bb(c):
            return jnp.stack([picked[b][c] for b in range(_BB)]).reshape(_BB, 1, 1)

        px1, py1, px2, py2, par = bb(0), bb(1), bb(2), bb(3), bb(4)

        xx1 = jnp.maximum(px1, x1)
        yy1 = jnp.maximum(py1, y1)
        xx2 = jnp.minimum(px2, x2)
        yy2 = jnp.minimum(py2, y2)
        inter = jnp.maximum(xx2 - xx1, 0.0) * jnp.maximum(yy2 - yy1, 0.0)
        iou = inter / (par + area - inter + 1e-9)
        kill = (iou > thr) | (flat == idx.reshape(_BB, 1, 1))
        ms = jnp.where(kill, -1.0, ms)

        hasf = has.astype(f32)[:, :, 0]
        rows = jnp.stack([
            jnp.stack([picked[b][0], picked[b][1],
                       picked[b][2], picked[b][3]] + zero8)
            for b in range(_BB)
        ])
        vals8 = rows[:, None, :] * hasf[:, :, None]
        outs[:, pl.ds(j, 1), :] = vals8
        return ms

    jax.lax.fori_loop(0, _POST_NMS_TOP_N, nms_body, ms0)

    col0 = (jax.lax.broadcasted_iota(i32, (_BB, _POST_NMS_TOP_N, 1), 0)
            + pl.program_id(0) * _BB).astype(f32)
    out_ref[...] = jnp.concatenate([col0, outs[:, 0:_POST_NMS_TOP_N, 0:4]],
                                   axis=2)


@jax.jit
def _impl(scores, bbox_deltas, im_info):
    sc = scores.transpose(0, 2, 3, 1).reshape(_B, _ROWS, _LANES)
    d = bbox_deltas.transpose(0, 2, 3, 1).reshape(_B, _N, 4)
    dx = d[..., 0].reshape(_B, _ROWS, _LANES)
    dy = d[..., 1].reshape(_B, _ROWS, _LANES)
    dw = d[..., 2].reshape(_B, _ROWS, _LANES)
    dh = d[..., 3].reshape(_B, _ROWS, _LANES)
    imf = jnp.broadcast_to(im_info[:, :2, None], (_B, 2, _LANES))
    imf = jnp.concatenate([imf, jnp.zeros((_B, 6, _LANES), jnp.float32)], axis=1)

    a1, a2, a3, a4 = (jnp.asarray(p) for p in _ANCH_PLANES)

    bspec = pl.BlockSpec((_BB, _ROWS, _LANES), lambda i: (i, 0, 0))
    aspec = pl.BlockSpec((_ROWS, _LANES), lambda i: (0, 0))
    return pl.pallas_call(
        _proposal_kernel,
        grid=(_G,),
        in_specs=[bspec] * 5 + [aspec] * 4
                 + [pl.BlockSpec((_BB, 8, _LANES), lambda i: (i, 0, 0))],
        out_specs=pl.BlockSpec((_BB, _POST_NMS_TOP_N, 5), lambda i: (i, 0, 0)),
        out_shape=jax.ShapeDtypeStruct((_B, _POST_NMS_TOP_N, 5), jnp.float32),
        scratch_shapes=[pltpu.VMEM((_BB, _ROWS, _LANES), jnp.float32)
                        for _ in range(5)]
                       + [pltpu.VMEM((_BB, 304, 8), jnp.float32)],
        compiler_params=pltpu.CompilerParams(
            dimension_semantics=("parallel",)),
    )(sc, dx, dy, dw, dh, a1, a2, a3, a4, imf)


def kernel(scores, bbox_deltas, im_info):
    return _impl(scores, bbox_deltas, im_info)
